# baseline (speedup 1.0000x reference)
import functools

import jax
import jax.numpy as jnp
from jax import lax
from jax.experimental import pallas as pl
from jax.experimental.pallas import tpu as pltpu
from jax.experimental.pallas import tpu_sc as plsc

F = 26
V = 100000
D = 32
B = 16384
BF = B * F

NC = 2
NS = 16
NW = NC * NS
L = 16

G = 128
GROUPS = BF // G
GROUPS_PER_W = GROUPS // NW
CHUNK_G = 8
NCHUNK = GROUPS_PER_W // CHUNK_G
CHUNK_ROWS = CHUNK_G * G


def _sc_body(idx_hbm, tab_hbm, out_hbm, idx_v, rows_v, sem):
    wid = lax.axis_index("s") * NC + lax.axis_index("c")
    g0 = wid * GROUPS_PER_W
    lane = lax.iota(jnp.int32, L)

    def chunk_body(c, carry):
        gbase = g0 + c * CHUNK_G
        pltpu.sync_copy(idx_hbm.at[pl.ds(gbase, CHUNK_G)], idx_v)

        def fixup(t, carry):
            j = t // (G // L)
            i = lax.rem(t, G // L)
            p0 = (gbase + j) * G + i * L
            fvec = lax.rem(lane + p0, F)
            sl = pl.ds(i * L, L)
            idx_v[j, sl] = idx_v[j, sl] + fvec * V
            return carry

        lax.fori_loop(0, CHUNK_G * (G // L), fixup, 0)

        copies = [
            pltpu.async_copy(tab_hbm.at[idx_v.at[j]], rows_v.at[j], sem)
            for j in range(CHUNK_G)
        ]
        for cp in copies:
            cp.wait()

        pltpu.sync_copy(rows_v, out_hbm.at[pl.ds(gbase, CHUNK_G)])
        return carry

    lax.fori_loop(0, NCHUNK, chunk_body, 0)


@jax.jit
def kernel(indices, tables):
    idx_flat = indices.reshape(GROUPS, G).astype(jnp.int32)
    tab_flat = tables.reshape(F * V, D)

    mesh = plsc.VectorSubcoreMesh(
        core_axis_name="c", subcore_axis_name="s",
        num_cores=NC, num_subcores=NS,
    )
    run = functools.partial(
        pl.kernel,
        mesh=mesh,
        out_type=jax.ShapeDtypeStruct((GROUPS, G, D), jnp.float32),
        scratch_types=[
            pltpu.VMEM((CHUNK_G, G), jnp.int32),
            pltpu.VMEM((CHUNK_G, G, D), jnp.float32),
            pltpu.SemaphoreType.DMA,
        ],
        compiler_params=pltpu.CompilerParams(use_tc_tiling_on_sc=False),
    )(_sc_body)
    out = run(idx_flat, tab_flat)
    return out.reshape(B, F, D)


# speedup vs baseline: 1.0085x; 1.0085x over previous
import functools

import jax
import jax.numpy as jnp
from jax import lax
from jax.experimental import pallas as pl
from jax.experimental.pallas import tpu as pltpu
from jax.experimental.pallas import tpu_sc as plsc

F = 26
V = 100000
D = 32
B = 16384
BF = B * F

NC = 2
NS = 16
NW = NC * NS
L = 16

ROWS_PER_W = BF // NW
K = 1024
NB = ROWS_PER_W // K
NBUF = 3


def _sc_body(idx_hbm, tab_hbm, out_hbm, idx_v, bufs, gsem, wsem):
    wid = lax.axis_index("s") * NC + lax.axis_index("c")
    r0 = wid * ROWS_PER_W
    lane = lax.iota(jnp.int32, L)

    pltpu.sync_copy(idx_hbm.at[pl.ds(r0, ROWS_PER_W)], idx_v)

    def fixup(t, carry):
        p0 = r0 + t * L
        fvec = lax.rem(lane + p0, F)
        sl = pl.ds(t * L, L)
        idx_v[sl] = idx_v[sl] + fvec * V
        return carry

    lax.fori_loop(0, ROWS_PER_W // L, fixup, 0)

    gather_handles = {}
    write_handles = {}

    def fire_gather(b):
        gather_handles[b] = pltpu.async_copy(
            tab_hbm.at[idx_v.at[pl.ds(b * K, K)]],
            bufs[b % NBUF],
            gsem.at[b % NBUF],
        )

    for b in range(min(NBUF - 1, NB)):
        fire_gather(b)

    for b in range(NB):
        gather_handles[b].wait()
        write_handles[b] = pltpu.async_copy(
            bufs[b % NBUF], out_hbm.at[pl.ds(r0 + b * K, K)], wsem.at[b % NBUF]
        )
        nxt = b + NBUF - 1
        if nxt < NB:
            if b >= 1:
                write_handles[b - 1].wait()
            fire_gather(nxt)

    for b in range(max(0, NB - (NBUF - 1)), NB):
        write_handles[b].wait()


@jax.jit
def kernel(indices, tables):
    idx_flat = indices.reshape(BF).astype(jnp.int32)
    tab_flat = tables.reshape(F * V, D)

    mesh = plsc.VectorSubcoreMesh(
        core_axis_name="c", subcore_axis_name="s",
        num_cores=NC, num_subcores=NS,
    )
    run = functools.partial(
        pl.kernel,
        mesh=mesh,
        out_type=jax.ShapeDtypeStruct((BF, D), jnp.float32),
        scratch_types=[
            pltpu.VMEM((ROWS_PER_W,), jnp.int32),
            [pltpu.VMEM((K, D), jnp.float32) for _ in range(NBUF)],
            pltpu.SemaphoreType.DMA((NBUF,)),
            pltpu.SemaphoreType.DMA((NBUF,)),
        ],
        compiler_params=pltpu.CompilerParams(use_tc_tiling_on_sc=False),
    )(_sc_body)
    out = run(idx_flat, tab_flat)
    return out.reshape(B, F, D)


# speedup vs baseline: 1.0138x; 1.0052x over previous
import functools

import jax
import jax.numpy as jnp
from jax import lax
from jax.experimental import pallas as pl
from jax.experimental.pallas import tpu as pltpu
from jax.experimental.pallas import tpu_sc as plsc

F = 26
V = 100000
D = 32
B = 16384

NC = 2
NS = 16
NW = NC * NS

BPW = B // NW
NBUF = 3


def _sc_body(idxt_hbm, tab_hbm, out_hbm, idx_v, bufs, gsem, wsem):
    wid = lax.axis_index("s") * NC + lax.axis_index("c")
    b0 = wid * BPW

    pltpu.sync_copy(idxt_hbm.at[:, pl.ds(b0, BPW)], idx_v)

    gather_handles = {}
    write_handles = {}

    def fire_gather(f):
        gather_handles[f] = pltpu.async_copy(
            tab_hbm.at[f].at[idx_v.at[f]],
            bufs[f % NBUF],
            gsem.at[f % NBUF],
        )

    for f in range(NBUF - 1):
        fire_gather(f)

    for f in range(F):
        gather_handles[f].wait()
        write_handles[f] = pltpu.async_copy(
            bufs[f % NBUF], out_hbm.at[pl.ds(b0, BPW), f], wsem.at[f % NBUF]
        )
        nxt = f + NBUF - 1
        if nxt < F:
            if f >= 1:
                write_handles[f - 1].wait()
            fire_gather(nxt)

    for f in range(F - (NBUF - 1), F):
        write_handles[f].wait()


@jax.jit
def kernel(indices, tables):
    idxt = indices.astype(jnp.int32).T

    mesh = plsc.VectorSubcoreMesh(
        core_axis_name="c", subcore_axis_name="s",
        num_cores=NC, num_subcores=NS,
    )
    run = functools.partial(
        pl.kernel,
        mesh=mesh,
        out_type=jax.ShapeDtypeStruct((B, F, D), jnp.float32),
        scratch_types=[
            pltpu.VMEM((F, BPW), jnp.int32),
            [pltpu.VMEM((BPW, D), jnp.float32) for _ in range(NBUF)],
            pltpu.SemaphoreType.DMA((NBUF,)),
            pltpu.SemaphoreType.DMA((NBUF,)),
        ],
        compiler_params=pltpu.CompilerParams(use_tc_tiling_on_sc=False),
    )(_sc_body)
    return run(idxt, tables)
